# Initial kernel scaffold; baseline (speedup 1.0000x reference)
import jax
import jax.numpy as jnp
from jax.experimental import pallas as pl


def kernel(batch, emb_weight):
    raise NotImplementedError("write your pallas kernel here")


# speedup vs baseline: 1.5366x; 1.5366x over previous
import functools

import jax
import jax.numpy as jnp
from jax import lax
from jax.experimental import pallas as pl
from jax.experimental.pallas import tpu as pltpu
from jax.experimental.pallas import tpu_sc as plsc

N_NODES = 100000
EMBED_DIM = 128
BATCH = 16384

N_CORES = 2
N_SUBCORES = 16
NW = N_CORES * N_SUBCORES
B_PER_W = BATCH // NW
CHUNK = 128
N_CHUNKS = B_PER_W // CHUNK

_mesh = plsc.VectorSubcoreMesh(core_axis_name="c", subcore_axis_name="s")


@functools.partial(
    pl.kernel,
    out_type=jax.ShapeDtypeStruct((BATCH, EMBED_DIM), jnp.float32),
    mesh=_mesh,
    scratch_types=[
        pltpu.VMEM((B_PER_W,), jnp.int32),
        [pltpu.VMEM((CHUNK, EMBED_DIM), jnp.float32) for _ in range(N_CHUNKS)],
        [pltpu.SemaphoreType.DMA for _ in range(N_CHUNKS)],
    ],
)
def _gather_kernel(table_hbm, idx_hbm, out_hbm, idx_v, rows, sems):
    wid = lax.axis_index("s") * N_CORES + lax.axis_index("c")
    base = wid * B_PER_W
    pltpu.sync_copy(idx_hbm.at[pl.ds(base, B_PER_W)], idx_v)
    copies = []
    for j in range(N_CHUNKS):
        copies.append(
            pltpu.async_copy(
                table_hbm.at[idx_v.at[pl.ds(j * CHUNK, CHUNK)]],
                rows[j],
                sems[j],
            )
        )
    for j in range(N_CHUNKS):
        copies[j].wait()
        pltpu.sync_copy(rows[j], out_hbm.at[pl.ds(base + j * CHUNK, CHUNK)])


def kernel(batch, emb_weight):
    return _gather_kernel(emb_weight, batch)


# speedup vs baseline: 1.5532x; 1.0108x over previous
import functools

import jax
import jax.numpy as jnp
from jax import lax
from jax.experimental import pallas as pl
from jax.experimental.pallas import tpu as pltpu
from jax.experimental.pallas import tpu_sc as plsc

N_NODES = 100000
EMBED_DIM = 128
BATCH = 16384

N_CORES = 2
N_SUBCORES = 16
NW = N_CORES * N_SUBCORES
B_PER_W = BATCH // NW
CHUNK = 128
N_CHUNKS = B_PER_W // CHUNK

_mesh = plsc.VectorSubcoreMesh(core_axis_name="c", subcore_axis_name="s")


@functools.partial(
    pl.kernel,
    out_type=jax.ShapeDtypeStruct((BATCH, EMBED_DIM), jnp.float32),
    mesh=_mesh,
    scratch_types=[
        pltpu.VMEM((B_PER_W,), jnp.int32),
        [pltpu.VMEM((CHUNK, EMBED_DIM), jnp.float32) for _ in range(N_CHUNKS)],
        [pltpu.SemaphoreType.DMA for _ in range(N_CHUNKS)],
        pltpu.SemaphoreType.DMA,
    ],
)
def _gather_kernel(table_hbm, idx_hbm, out_hbm, idx_v, rows, sems, wsem):
    wid = lax.axis_index("s") * N_CORES + lax.axis_index("c")
    base = wid * B_PER_W
    pltpu.sync_copy(idx_hbm.at[pl.ds(base, B_PER_W)], idx_v)
    copies = []
    for j in range(N_CHUNKS):
        copies.append(
            pltpu.async_copy(
                table_hbm.at[idx_v.at[pl.ds(j * CHUNK, CHUNK)]],
                rows[j],
                sems[j],
            )
        )
    writes = []
    for j in range(N_CHUNKS):
        copies[j].wait()
        writes.append(
            pltpu.async_copy(
                rows[j], out_hbm.at[pl.ds(base + j * CHUNK, CHUNK)], wsem
            )
        )
    for w in writes:
        w.wait()


def kernel(batch, emb_weight):
    return _gather_kernel(emb_weight, batch)


# speedup vs baseline: 1.5547x; 1.0009x over previous
import functools

import jax
import jax.numpy as jnp
from jax import lax
from jax.experimental import pallas as pl
from jax.experimental.pallas import tpu as pltpu
from jax.experimental.pallas import tpu_sc as plsc

N_NODES = 100000
EMBED_DIM = 128
BATCH = 16384

N_CORES = 2
N_SUBCORES = 16
NW = N_CORES * N_SUBCORES
B_PER_W = BATCH // NW
CHUNK = 128
N_CHUNKS = B_PER_W // CHUNK

_mesh = plsc.VectorSubcoreMesh(core_axis_name="c", subcore_axis_name="s")


@functools.partial(
    pl.kernel,
    out_type=jax.ShapeDtypeStruct((BATCH, EMBED_DIM), jnp.float32),
    mesh=_mesh,
    scratch_types=[
        pltpu.VMEM((B_PER_W,), jnp.int32),
        [pltpu.VMEM((CHUNK, EMBED_DIM), jnp.float32) for _ in range(N_CHUNKS)],
        [pltpu.SemaphoreType.DMA for _ in range(N_CHUNKS)],
        [pltpu.SemaphoreType.DMA for _ in range(N_CHUNKS)],
        pltpu.SemaphoreType.DMA,
    ],
)
def _gather_kernel(table_hbm, idx_hbm, out_hbm, idx_v, rows, isems, gsems, wsem):
    wid = lax.axis_index("s") * N_CORES + lax.axis_index("c")
    base = wid * B_PER_W
    idx_copies = []
    for j in range(N_CHUNKS):
        idx_copies.append(
            pltpu.async_copy(
                idx_hbm.at[pl.ds(base + j * CHUNK, CHUNK)],
                idx_v.at[pl.ds(j * CHUNK, CHUNK)],
                isems[j],
            )
        )
    gathers = []
    for j in range(N_CHUNKS):
        idx_copies[j].wait()
        gathers.append(
            pltpu.async_copy(
                table_hbm.at[idx_v.at[pl.ds(j * CHUNK, CHUNK)]],
                rows[j],
                gsems[j],
            )
        )
    writes = []
    for j in range(N_CHUNKS):
        gathers[j].wait()
        writes.append(
            pltpu.async_copy(
                rows[j], out_hbm.at[pl.ds(base + j * CHUNK, CHUNK)], wsem
            )
        )
    for w in writes:
        w.wait()


def kernel(batch, emb_weight):
    return _gather_kernel(emb_weight, batch)


# speedup vs baseline: 1.5647x; 1.0065x over previous
import functools

import jax
import jax.numpy as jnp
from jax import lax
from jax.experimental import pallas as pl
from jax.experimental.pallas import tpu as pltpu
from jax.experimental.pallas import tpu_sc as plsc

N_NODES = 100000
EMBED_DIM = 128
BATCH = 16384

N_CORES = 2
N_SUBCORES = 16
NW = N_CORES * N_SUBCORES
B_PER_W = BATCH // NW
CHUNK = 256
N_CHUNKS = B_PER_W // CHUNK

_mesh = plsc.VectorSubcoreMesh(core_axis_name="c", subcore_axis_name="s")


@functools.partial(
    pl.kernel,
    out_type=jax.ShapeDtypeStruct((BATCH, EMBED_DIM), jnp.float32),
    mesh=_mesh,
    scratch_types=[
        pltpu.VMEM((B_PER_W,), jnp.int32),
        [pltpu.VMEM((CHUNK, EMBED_DIM), jnp.float32) for _ in range(N_CHUNKS)],
        [pltpu.SemaphoreType.DMA for _ in range(N_CHUNKS)],
        [pltpu.SemaphoreType.DMA for _ in range(N_CHUNKS)],
        pltpu.SemaphoreType.DMA,
    ],
)
def _gather_kernel(table_hbm, idx_hbm, out_hbm, idx_v, rows, isems, gsems, wsem):
    wid = lax.axis_index("s") * N_CORES + lax.axis_index("c")
    base = wid * B_PER_W
    idx_copies = []
    for j in range(N_CHUNKS):
        idx_copies.append(
            pltpu.async_copy(
                idx_hbm.at[pl.ds(base + j * CHUNK, CHUNK)],
                idx_v.at[pl.ds(j * CHUNK, CHUNK)],
                isems[j],
            )
        )
    gathers = []
    for j in range(N_CHUNKS):
        idx_copies[j].wait()
        gathers.append(
            pltpu.async_copy(
                table_hbm.at[idx_v.at[pl.ds(j * CHUNK, CHUNK)]],
                rows[j],
                gsems[j],
            )
        )
    writes = []
    for j in range(N_CHUNKS):
        gathers[j].wait()
        writes.append(
            pltpu.async_copy(
                rows[j], out_hbm.at[pl.ds(base + j * CHUNK, CHUNK)], wsem
            )
        )
    for w in writes:
        w.wait()


def kernel(batch, emb_weight):
    return _gather_kernel(emb_weight, batch)


# speedup vs baseline: 1.5794x; 1.0094x over previous
import functools

import jax
import jax.numpy as jnp
from jax import lax
from jax.experimental import pallas as pl
from jax.experimental.pallas import tpu as pltpu
from jax.experimental.pallas import tpu_sc as plsc

N_NODES = 100000
EMBED_DIM = 128
BATCH = 16384

N_CORES = 2
N_SUBCORES = 16
NW = N_CORES * N_SUBCORES
B_PER_W = BATCH // NW
CHUNK = 512
N_CHUNKS = B_PER_W // CHUNK

_mesh = plsc.VectorSubcoreMesh(core_axis_name="c", subcore_axis_name="s")


@functools.partial(
    pl.kernel,
    out_type=jax.ShapeDtypeStruct((BATCH, EMBED_DIM), jnp.float32),
    mesh=_mesh,
    scratch_types=[
        pltpu.VMEM((B_PER_W,), jnp.int32),
        [pltpu.VMEM((CHUNK, EMBED_DIM), jnp.float32) for _ in range(N_CHUNKS)],
        [pltpu.SemaphoreType.DMA for _ in range(N_CHUNKS)],
        [pltpu.SemaphoreType.DMA for _ in range(N_CHUNKS)],
        pltpu.SemaphoreType.DMA,
    ],
)
def _gather_kernel(table_hbm, idx_hbm, out_hbm, idx_v, rows, isems, gsems, wsem):
    wid = lax.axis_index("s") * N_CORES + lax.axis_index("c")
    base = wid * B_PER_W
    idx_copies = []
    for j in range(N_CHUNKS):
        idx_copies.append(
            pltpu.async_copy(
                idx_hbm.at[pl.ds(base + j * CHUNK, CHUNK)],
                idx_v.at[pl.ds(j * CHUNK, CHUNK)],
                isems[j],
            )
        )
    gathers = []
    for j in range(N_CHUNKS):
        idx_copies[j].wait()
        gathers.append(
            pltpu.async_copy(
                table_hbm.at[idx_v.at[pl.ds(j * CHUNK, CHUNK)]],
                rows[j],
                gsems[j],
            )
        )
    writes = []
    for j in range(N_CHUNKS):
        gathers[j].wait()
        writes.append(
            pltpu.async_copy(
                rows[j], out_hbm.at[pl.ds(base + j * CHUNK, CHUNK)], wsem
            )
        )
    for w in writes:
        w.wait()


def kernel(batch, emb_weight):
    return _gather_kernel(emb_weight, batch)
